# Initial kernel scaffold; baseline (speedup 1.0000x reference)
import jax
import jax.numpy as jnp
from jax.experimental import pallas as pl


def kernel(inputs, emb, Wih0, Whh0, bih0, bhh0, Wih1, Whh1, bih1, bhh1, Wih2, Whh2, bih2, bhh2, Wih3, Whh3, bih3, bhh3, Wout, bout):
    raise NotImplementedError("write your pallas kernel here")


# speedup vs baseline: 2.5483x; 2.5483x over previous
import functools

import jax
import jax.numpy as jnp
from jax import lax
from jax.experimental import pallas as pl
from jax.experimental.pallas import tpu as pltpu
from jax.experimental.pallas import tpu_sc as plsc

V = 1000
E = 128
H = 512
C = 128
B = 1024
T = 50


_NC = 2
_NS = 16
_NW = _NC * _NS
_N = T * B
_B_PER_W = _N // _NW
_CH = 128
_N_FULL = _B_PER_W // _CH
_REM = _B_PER_W - _N_FULL * _CH

def _sc_gather_body(idx_hbm, table_hbm, out_hbm, idx_v, buf0, buf1, sem0, sem1):
    wid = lax.axis_index("s") * _NC + lax.axis_index("c")
    base = wid * _B_PER_W
    pltpu.sync_copy(idx_hbm.at[pl.ds(base, _B_PER_W)], idx_v)

    bufs = (buf0, buf1)
    sems = (sem0, sem1)
    copies = [None, None]
    for k in range(_N_FULL):
        s = k % 2
        copies[s] = pltpu.async_copy(
            table_hbm.at[idx_v.at[pl.ds(k * _CH, _CH)]], bufs[s], sems[s])
        if k > 0:
            prev = (k - 1) % 2
            copies[prev].wait()
            pltpu.sync_copy(bufs[prev],
                            out_hbm.at[pl.ds(base + (k - 1) * _CH, _CH)])
    last = (_N_FULL - 1) % 2
    copies[last].wait()
    pltpu.sync_copy(bufs[last],
                    out_hbm.at[pl.ds(base + (_N_FULL - 1) * _CH, _CH)])
    if _REM:
        rbuf = bufs[(_N_FULL) % 2]
        pltpu.async_copy(
            table_hbm.at[idx_v.at[pl.ds(_N_FULL * _CH, _REM)]],
            rbuf.at[pl.ds(0, _REM)], sems[(_N_FULL) % 2]).wait()
        pltpu.sync_copy(rbuf.at[pl.ds(0, _REM)],
                        out_hbm.at[pl.ds(base + _N_FULL * _CH, _REM)])


@functools.cache
def _sc_gather():
    mesh = plsc.VectorSubcoreMesh(core_axis_name="c", subcore_axis_name="s")
    return pl.kernel(
        _sc_gather_body,
        mesh=mesh,
        out_type=jax.ShapeDtypeStruct((_N, E), jnp.float32),
        scratch_types=[
            pltpu.VMEM((_B_PER_W,), jnp.int32),
            pltpu.VMEM((_CH, E), jnp.float32),
            pltpu.VMEM((_CH, E), jnp.float32),
            pltpu.SemaphoreType.DMA,
            pltpu.SemaphoreType.DMA,
        ],
    )


_BB = 256
_TCC = 5


def _rnn_body(x0_ref, w0, a0, b0, w1, a1, b1, w2, a2, b2, w3, a3, b3,
              wout, bout, out_ref, xbuf):

    def proj(src_ref, w_ref, b_ref, in_dim):
        w = w_ref[...]
        b = b_ref[...]

        def chunk(i, _):
            xc = src_ref[pl.ds(i * _TCC, _TCC)]
            xc2 = xc.reshape(_TCC * _BB, in_dim)
            p = jnp.dot(xc2, w, preferred_element_type=jnp.float32) + b
            xbuf[pl.ds(i * _TCC, _TCC)] = p.reshape(_TCC, _BB, H)
            return 0

        lax.fori_loop(0, T // _TCC, chunk, 0)

    def scan(a_ref, store):
        a = a_ref[...]

        def step(t, carry):
            h, s = carry
            h = jnp.tanh(xbuf[t] +
                         jnp.dot(h, a, preferred_element_type=jnp.float32))
            if store:
                xbuf[t] = h
            return h, s + h

        z = jnp.zeros((_BB, H), jnp.float32)
        _, s = lax.fori_loop(0, T, step, (z, z))
        return s

    proj(x0_ref, w0, b0, E)
    scan(a0, True)
    for w_ref, a_ref, b_ref, last in ((w1, a1, b1, False),
                                      (w2, a2, b2, False),
                                      (w3, a3, b3, True)):
        proj(xbuf, w_ref, b_ref, H)
        s = scan(a_ref, not last)

    hmean = s * (1.0 / T)
    out_ref[...] = (jnp.dot(hmean, wout[...], preferred_element_type=jnp.float32)
                    + bout[...])


def _full(shape):
    return pl.BlockSpec(shape, lambda i: (0,) * len(shape))


_rnn_fwd = pl.pallas_call(
    _rnn_body,
    grid=(B // _BB,),
    in_specs=[
        pl.BlockSpec((T, _BB, E), lambda i: (0, i, 0)),
        _full((E, H)), _full((H, H)), _full((1, H)),
        _full((H, H)), _full((H, H)), _full((1, H)),
        _full((H, H)), _full((H, H)), _full((1, H)),
        _full((H, H)), _full((H, H)), _full((1, H)),
        _full((H, C)), _full((1, C)),
    ],
    out_specs=pl.BlockSpec((_BB, C), lambda i: (i, 0)),
    out_shape=jax.ShapeDtypeStruct((B, C), jnp.float32),
    scratch_shapes=[pltpu.VMEM((T, _BB, H), jnp.float32)],
    compiler_params=pltpu.CompilerParams(
        dimension_semantics=("parallel",),
    ),
)


def kernel(inputs, emb, Wih0, Whh0, bih0, bhh0, Wih1, Whh1, bih1, bhh1,
           Wih2, Whh2, bih2, bhh2, Wih3, Whh3, bih3, bhh3, Wout, bout):
    idx = inputs.astype(jnp.int32).T.reshape(-1)
    x0 = _sc_gather()(idx, emb).reshape(T, B, E)
    args = [x0]
    for Wih, Whh, bih, bhh in ((Wih0, Whh0, bih0, bhh0),
                               (Wih1, Whh1, bih1, bhh1),
                               (Wih2, Whh2, bih2, bhh2),
                               (Wih3, Whh3, bih3, bhh3)):
        args += [Wih.T, Whh.T, (bih + bhh).reshape(1, H)]
    args += [Wout.T, bout.reshape(1, C)]
    return _rnn_fwd(*args)
